# baseline (speedup 1.0000x reference)
import jax
import jax.numpy as jnp
from jax import lax
from jax.experimental import pallas as pl
from jax.experimental.pallas import tpu as pltpu
from jax.experimental.pallas import tpu_sc as plsc

ALPHA = 1.0
BETA = 10.0
OHEM_RATIO = 3.0
SMOOTH = 1.0

B, H, W = 16, 512, 512
N = B * H * W
NC, NS, L = 2, 16, 16
NW = NC * NS
HPW = H // 2

RB = 16
NCH = HPW // RB
NBUF = 4
VECS_PER_ROW = W // L

MB = 6
SH = 23 - MB
TOPKEY = 127 << MB
NBINS = 2048
HISTW = NBINS * L
HROWS = HISTW // 128
TAILBITS = (TOPKEY - (NBINS - 1) + 1) << SH


def _sc_hist_body(p_hbm, t_hbm, hist_out,
                  pb0, pb1, pb2, pb3, tb0, tb1, tb2, tb3, hist,
                  sp0, sp1, sp2, sp3, st0, st1, st2, st3):
    c = lax.axis_index("c")
    s = lax.axis_index("s")
    wid = s * NC + c
    img = wid >> 1
    row0 = (wid & 1) * HPW

    zero16 = jnp.zeros((L,), jnp.float32)
    ones16 = jnp.ones((L,), jnp.float32)
    lane = lax.iota(jnp.int32, L)

    def zstep(i, carry):
        hist[i, pl.ds(0, L)] = zero16
        hist[i, pl.ds(16, L)] = zero16
        hist[i, pl.ds(32, L)] = zero16
        hist[i, pl.ds(48, L)] = zero16
        hist[i, pl.ds(64, L)] = zero16
        hist[i, pl.ds(80, L)] = zero16
        hist[i, pl.ds(96, L)] = zero16
        hist[i, pl.ds(112, L)] = zero16
        return carry

    lax.fori_loop(0, HROWS, zstep, 0)

    pbufs = (pb0, pb1, pb2, pb3)
    tbufs = (tb0, tb1, tb2, tb3)
    psems = (sp0, sp1, sp2, sp3)
    tsems = (st0, st1, st2, st3)

    def start(ci, slot):
        r = row0 + ci * RB
        pltpu.async_copy(p_hbm.at[img, 0, pl.ds(r, RB), :], pbufs[slot], psems[slot])
        pltpu.async_copy(t_hbm.at[img, 0, pl.ds(r, RB), :], tbufs[slot], tsems[slot])

    def wait(slot):
        pltpu.make_async_copy(p_hbm.at[0, 0, pl.ds(0, RB), :], pbufs[slot], psems[slot]).wait()
        pltpu.make_async_copy(t_hbm.at[0, 0, pl.ds(0, RB), :], tbufs[slot], tsems[slot]).wait()

    for pre in range(NBUF - 1):
        start(pre, pre)

    def chunk(ci, slot):
        @pl.when(ci + NBUF - 1 < NCH)
        def _():
            start(ci + NBUF - 1, (slot + NBUF - 1) % NBUF)

        wait(slot)
        pb = pbufs[slot]
        tb = tbufs[slot]

        def inner(r, carry):
            for g in range(VECS_PER_ROW // 8):
                ps = [pb[r, pl.ds((g * 8 + u) * L, L)] for u in range(8)]
                ts = [tb[r, pl.ds((g * 8 + u) * L, L)] for u in range(8)]
                hrs, hcols = [], []
                for u in range(8):
                    qeff = jnp.maximum(1.0 - ps[u], ts[u])
                    bits = plsc.bitcast(qeff, jnp.int32)
                    raw = (TOPKEY - (bits >> SH)).astype(jnp.uint32)
                    bin_ = jnp.minimum(raw, jnp.uint32(NBINS - 1)).astype(jnp.int32)
                    hrs.append(bin_ >> 3)
                    hcols.append(((bin_ & 7) << 4) + lane)
                for u in range(8):
                    plsc.addupdate_scatter(hist, [hrs[u], hcols[u]], ones16)
            return carry

        lax.fori_loop(0, RB, inner, 0)

    def outer(g, carry):
        for b in range(NBUF):
            chunk(g * NBUF + b, b)
        return carry

    lax.fori_loop(0, NCH // NBUF, outer, 0)

    pltpu.sync_copy(hist, hist_out.at[pl.ds(wid * HROWS, HROWS), :])


def _sc_hist(p4, t4):
    mesh = plsc.VectorSubcoreMesh(core_axis_name="c", subcore_axis_name="s")
    return pl.kernel(
        _sc_hist_body,
        out_type=jax.ShapeDtypeStruct((NW * HROWS, 128), jnp.float32),
        mesh=mesh,
        compiler_params=pltpu.CompilerParams(needs_layout_passes=False,
                                             skip_device_barrier=True),
        scratch_types=(
            [pltpu.VMEM((RB, W), jnp.float32)] * (2 * NBUF)
            + [pltpu.VMEM((HROWS, 128), jnp.float32)]
            + [pltpu.SemaphoreType.DMA] * (2 * NBUF)
        ),
    )(p4, t4)


def _tc_dense_body(p_ref, t_ref, bp_ref, tp_ref, tg_ref, m_ref, out_ref):
    p = p_ref[...]
    t = t_ref[...]
    bp = bp_ref[...]
    tp = tp_ref[...]
    tg = tg_ref[...]
    m = m_ref[...]

    lp = jnp.maximum(jnp.log(p), -100.0)
    q = 1.0 - p
    qeff = jnp.maximum(q, t)
    v = -jnp.maximum(jnp.log(qeff), -100.0)

    qbits = lax.bitcast_convert_type(qeff, jnp.int32)
    tail = qbits < TAILBITS

    n_pos = jnp.sum(t)
    s_pos = jnp.sum(t * (-lp))
    s_neg = jnp.sum(v)
    s_tail = jnp.sum(jnp.where(tail, v, 0.0))
    c_tail = jnp.sum(jnp.where(tail, 1.0, 0.0))
    s_bp = jnp.sum(bp)
    s_inter = jnp.sum(bp * t)
    s_l1 = jnp.sum(jnp.abs(tp - tg) * m)
    s_m = jnp.sum(m)

    vals = [n_pos, s_pos, s_neg, s_tail, c_tail, s_bp, s_inter, s_l1, s_m]

    @pl.when(pl.program_id(0) == 0)
    def _():
        for i, val in enumerate(vals):
            out_ref[0, i] = val

    @pl.when(pl.program_id(0) != 0)
    def _():
        for i, val in enumerate(vals):
            out_ref[0, i] += val


def _tc_dense(p4, t4, bp4, tp4, tg4, m4):
    spec = pl.BlockSpec((4, 1, H, W), lambda i: (i, 0, 0, 0))
    return pl.pallas_call(
        _tc_dense_body,
        grid=(B // 4,),
        in_specs=[spec] * 6,
        out_specs=pl.BlockSpec(memory_space=pltpu.SMEM),
        out_shape=jax.ShapeDtypeStruct((1, 16), jnp.float32),
    )(p4, t4, bp4, tp4, tg4, m4)


def _tc_fin_body(sc_ref, hc_ref, out_ref):
    n_pos = sc_ref[0, 0]
    s_pos = sc_ref[0, 1]
    s_neg = sc_ref[0, 2]
    s_tail = sc_ref[0, 3]
    c_tail = sc_ref[0, 4]
    s_bp = sc_ref[0, 5]
    s_inter = sc_ref[0, 6]
    s_l1 = sc_ref[0, 7]
    s_m = sc_ref[0, 8]

    n_neg = jnp.float32(N) - n_pos
    k = jnp.minimum(n_neg, jnp.floor(n_pos * OHEM_RATIO))

    hc = jnp.sum(hc_ref[...].reshape(NW, HROWS, 128), axis=0)
    r_i = lax.broadcasted_iota(jnp.int32, hc.shape, 0)
    c_i = lax.broadcasted_iota(jnp.int32, hc.shape, 1)
    binv = lax.shift_right_logical(r_i * 128 + c_i, 4)

    shex = TOPKEY - binv
    lo_q = lax.bitcast_convert_type(lax.shift_left(shex, SH), jnp.float32)
    hi_q = lax.bitcast_convert_type(lax.shift_left(shex + 1, SH), jnp.float32)
    vbar = 1.0 - (hi_q * jnp.log(hi_q) - lo_q * jnp.log(lo_q)) / (hi_q - lo_q)
    vbar = jnp.where(binv == 0, 0.0, vbar)

    def bstep(i, state):
        lo, hi = state
        mid = lax.div(lo + hi, 2)
        cnt = jnp.sum(jnp.where(binv >= mid, hc, 0.0))
        sel = cnt >= k
        return jnp.where(sel, mid, lo), jnp.where(sel, hi, mid)

    bstar, _ = lax.fori_loop(0, 12, bstep, (jnp.int32(0), jnp.int32(NBINS)))

    above = binv > bstar
    c_above = jnp.sum(jnp.where(above, hc, 0.0))
    regular_above = above & (binv < NBINS - 1)
    s_above = jnp.sum(jnp.where(regular_above, hc * vbar, 0.0))
    s_above = s_above + jnp.where(bstar < NBINS - 1, s_tail, 0.0)

    r = k - c_above
    vtail_avg = s_tail / jnp.maximum(c_tail, 1.0)
    in_bin = jnp.sum(jnp.where(binv == bstar, hc * vbar, 0.0))
    cnt_bin = jnp.sum(jnp.where(binv == bstar, hc, 0.0))
    vbar_bstar = in_bin / jnp.maximum(cnt_bin, 1.0)
    est = r * jnp.where(bstar == NBINS - 1, vtail_avg, vbar_bstar)

    sum_topk = jnp.where(k >= n_neg, s_neg, s_above + est)

    pos_loss = s_pos / (n_pos + 1e-6)
    neg_loss = sum_topk / k
    dice = (2.0 * s_inter + SMOOTH) / (s_bp + n_pos + SMOOTH)
    loss_binary = 1.0 - dice
    loss_thresh = s_l1 / (s_m + 1e-6)
    out_ref[0, 0] = pos_loss + neg_loss + ALPHA * loss_binary + BETA * loss_thresh


def _tc_finalize(scalars, hc_all):
    return pl.pallas_call(
        _tc_fin_body,
        in_specs=[
            pl.BlockSpec(memory_space=pltpu.SMEM),
            pl.BlockSpec(memory_space=pltpu.VMEM),
        ],
        out_specs=pl.BlockSpec(memory_space=pltpu.SMEM),
        out_shape=jax.ShapeDtypeStruct((1, 1), jnp.float32),
    )(scalars, hc_all)


def kernel(prob_map_pred, thresh_map_pred, binary_map_pred,
           prob_map_gt, thresh_map_gt, mask_gt):
    hc_all = _sc_hist(prob_map_pred, prob_map_gt)
    scalars = _tc_dense(prob_map_pred, prob_map_gt, binary_map_pred,
                        thresh_map_pred, thresh_map_gt, mask_gt)
    out = _tc_finalize(scalars, hc_all)
    return out[0, 0]


# speedup vs baseline: 1.0346x; 1.0346x over previous
import jax
import jax.numpy as jnp
from jax import lax
from jax.experimental import pallas as pl
from jax.experimental.pallas import tpu as pltpu
from jax.experimental.pallas import tpu_sc as plsc

ALPHA = 1.0
BETA = 10.0
OHEM_RATIO = 3.0
SMOOTH = 1.0

B, H, W = 16, 512, 512
N = B * H * W
NC, NS, L = 2, 16, 16
NW = NC * NS
HPW = H // 2

RB = 16
NCH = HPW // RB
NBUF = 4
VECS_PER_ROW = W // L

MB = 6
SH = 23 - MB
TOPKEY = 127 << MB
NBINS = 2048
HISTW = NBINS * L
HROWS = HISTW // 128
TAILBITS = (TOPKEY - (NBINS - 1) + 1) << SH


def _sc_hist_body(p_hbm, t_hbm, hist_out,
                  pb0, pb1, pb2, pb3, tb0, tb1, tb2, tb3, hist,
                  sp0, sp1, sp2, sp3, st0, st1, st2, st3):
    c = lax.axis_index("c")
    s = lax.axis_index("s")
    wid = s * NC + c
    img = wid >> 1
    row0 = (wid & 1) * HPW

    zero16 = jnp.zeros((L,), jnp.float32)
    ones16 = jnp.ones((L,), jnp.float32)
    lane = lax.iota(jnp.int32, L)

    def zstep(i, carry):
        hist[i, pl.ds(0, L)] = zero16
        hist[i, pl.ds(16, L)] = zero16
        hist[i, pl.ds(32, L)] = zero16
        hist[i, pl.ds(48, L)] = zero16
        hist[i, pl.ds(64, L)] = zero16
        hist[i, pl.ds(80, L)] = zero16
        hist[i, pl.ds(96, L)] = zero16
        hist[i, pl.ds(112, L)] = zero16
        return carry

    lax.fori_loop(0, HROWS, zstep, 0)

    pbufs = (pb0, pb1, pb2, pb3)
    tbufs = (tb0, tb1, tb2, tb3)
    psems = (sp0, sp1, sp2, sp3)
    tsems = (st0, st1, st2, st3)

    def start(ci, slot):
        r = row0 + ci * RB
        pltpu.async_copy(p_hbm.at[img, 0, pl.ds(r, RB), :], pbufs[slot], psems[slot])
        pltpu.async_copy(t_hbm.at[img, 0, pl.ds(r, RB), :], tbufs[slot], tsems[slot])

    def wait(slot):
        pltpu.make_async_copy(p_hbm.at[0, 0, pl.ds(0, RB), :], pbufs[slot], psems[slot]).wait()
        pltpu.make_async_copy(t_hbm.at[0, 0, pl.ds(0, RB), :], tbufs[slot], tsems[slot]).wait()

    for pre in range(NBUF - 1):
        start(pre, pre)

    def chunk(ci, slot):
        @pl.when(ci + NBUF - 1 < NCH)
        def _():
            start(ci + NBUF - 1, (slot + NBUF - 1) % NBUF)

        wait(slot)
        pb = pbufs[slot]
        tb = tbufs[slot]

        def inner(r, carry):
            for g in range(VECS_PER_ROW // 8):
                ps = [pb[r, pl.ds((g * 8 + u) * L, L)] for u in range(8)]
                ts = [tb[r, pl.ds((g * 8 + u) * L, L)] for u in range(8)]
                hrs, hcols = [], []
                for u in range(8):
                    qeff = jnp.maximum(1.0 - ps[u], ts[u])
                    bits = plsc.bitcast(qeff, jnp.int32)
                    raw = (TOPKEY - (bits >> SH)).astype(jnp.uint32)
                    bin_ = jnp.minimum(raw, jnp.uint32(NBINS - 1)).astype(jnp.int32)
                    hrs.append(bin_ >> 3)
                    hcols.append(((bin_ & 7) << 4) + lane)
                for u in range(8):
                    plsc.addupdate_scatter(hist, [hrs[u], hcols[u]], ones16)
            return carry

        lax.fori_loop(0, RB, inner, 0)

    def outer(g, carry):
        for b in range(NBUF):
            chunk(g * NBUF + b, b)
        return carry

    lax.fori_loop(0, NCH // NBUF, outer, 0)

    pltpu.sync_copy(hist, hist_out.at[pl.ds(wid * HROWS, HROWS), :])


def _sc_hist(p4, t4):
    mesh = plsc.VectorSubcoreMesh(core_axis_name="c", subcore_axis_name="s")
    return pl.kernel(
        _sc_hist_body,
        out_type=jax.ShapeDtypeStruct((NW * HROWS, 128), jnp.float32),
        mesh=mesh,
        compiler_params=pltpu.CompilerParams(needs_layout_passes=False,
                                             skip_device_barrier=True),
        scratch_types=(
            [pltpu.VMEM((RB, W), jnp.float32)] * (2 * NBUF)
            + [pltpu.VMEM((HROWS, 128), jnp.float32)]
            + [pltpu.SemaphoreType.DMA] * (2 * NBUF)
        ),
    )(p4, t4)


def _tc_dense_body(p_ref, t_ref, bp_ref, tp_ref, tg_ref, m_ref, out_ref):
    p = p_ref[...]
    t = t_ref[...]
    bp = bp_ref[...]
    tp = tp_ref[...]
    tg = tg_ref[...]
    m = m_ref[...]

    lp = jnp.maximum(jnp.log(p), -100.0)
    q = 1.0 - p
    qeff = jnp.maximum(q, t)
    v = -jnp.maximum(jnp.log(qeff), -100.0)

    qbits = lax.bitcast_convert_type(qeff, jnp.int32)
    tail = qbits < TAILBITS

    n_pos = jnp.sum(t)
    s_pos = jnp.sum(t * (-lp))
    s_neg = jnp.sum(v)
    s_tail = jnp.sum(jnp.where(tail, v, 0.0))
    c_tail = jnp.sum(jnp.where(tail, 1.0, 0.0))
    s_bp = jnp.sum(bp)
    s_inter = jnp.sum(bp * t)
    s_l1 = jnp.sum(jnp.abs(tp - tg) * m)
    s_m = jnp.sum(m)

    vals = [n_pos, s_pos, s_neg, s_tail, c_tail, s_bp, s_inter, s_l1, s_m]

    @pl.when(pl.program_id(0) == 0)
    def _():
        for i, val in enumerate(vals):
            out_ref[0, i] = val

    @pl.when(pl.program_id(0) != 0)
    def _():
        for i, val in enumerate(vals):
            out_ref[0, i] += val


def _tc_dense(p4, t4, bp4, tp4, tg4, m4):
    spec = pl.BlockSpec((2, 1, H, W), lambda i: (i, 0, 0, 0))
    return pl.pallas_call(
        _tc_dense_body,
        grid=(B // 2,),
        in_specs=[spec] * 6,
        out_specs=pl.BlockSpec(memory_space=pltpu.SMEM),
        out_shape=jax.ShapeDtypeStruct((1, 16), jnp.float32),
    )(p4, t4, bp4, tp4, tg4, m4)


def _tc_fin_body(sc_ref, hc_ref, out_ref):
    n_pos = sc_ref[0, 0]
    s_pos = sc_ref[0, 1]
    s_neg = sc_ref[0, 2]
    s_tail = sc_ref[0, 3]
    c_tail = sc_ref[0, 4]
    s_bp = sc_ref[0, 5]
    s_inter = sc_ref[0, 6]
    s_l1 = sc_ref[0, 7]
    s_m = sc_ref[0, 8]

    n_neg = jnp.float32(N) - n_pos
    k = jnp.minimum(n_neg, jnp.floor(n_pos * OHEM_RATIO))

    hc = jnp.sum(hc_ref[...].reshape(NW, HROWS, 128), axis=0)
    r_i = lax.broadcasted_iota(jnp.int32, hc.shape, 0)
    c_i = lax.broadcasted_iota(jnp.int32, hc.shape, 1)
    binv = lax.shift_right_logical(r_i * 128 + c_i, 4)

    shex = TOPKEY - binv
    lo_q = lax.bitcast_convert_type(lax.shift_left(shex, SH), jnp.float32)
    hi_q = lax.bitcast_convert_type(lax.shift_left(shex + 1, SH), jnp.float32)
    vbar = 1.0 - (hi_q * jnp.log(hi_q) - lo_q * jnp.log(lo_q)) / (hi_q - lo_q)
    vbar = jnp.where(binv == 0, 0.0, vbar)

    def bstep(i, state):
        lo, hi = state
        mid = lax.div(lo + hi, 2)
        cnt = jnp.sum(jnp.where(binv >= mid, hc, 0.0))
        sel = cnt >= k
        return jnp.where(sel, mid, lo), jnp.where(sel, hi, mid)

    bstar, _ = lax.fori_loop(0, 12, bstep, (jnp.int32(0), jnp.int32(NBINS)))

    above = binv > bstar
    c_above = jnp.sum(jnp.where(above, hc, 0.0))
    regular_above = above & (binv < NBINS - 1)
    s_above = jnp.sum(jnp.where(regular_above, hc * vbar, 0.0))
    s_above = s_above + jnp.where(bstar < NBINS - 1, s_tail, 0.0)

    r = k - c_above
    vtail_avg = s_tail / jnp.maximum(c_tail, 1.0)
    in_bin = jnp.sum(jnp.where(binv == bstar, hc * vbar, 0.0))
    cnt_bin = jnp.sum(jnp.where(binv == bstar, hc, 0.0))
    vbar_bstar = in_bin / jnp.maximum(cnt_bin, 1.0)
    est = r * jnp.where(bstar == NBINS - 1, vtail_avg, vbar_bstar)

    sum_topk = jnp.where(k >= n_neg, s_neg, s_above + est)

    pos_loss = s_pos / (n_pos + 1e-6)
    neg_loss = sum_topk / k
    dice = (2.0 * s_inter + SMOOTH) / (s_bp + n_pos + SMOOTH)
    loss_binary = 1.0 - dice
    loss_thresh = s_l1 / (s_m + 1e-6)
    out_ref[0, 0] = pos_loss + neg_loss + ALPHA * loss_binary + BETA * loss_thresh


def _tc_finalize(scalars, hc_all):
    return pl.pallas_call(
        _tc_fin_body,
        in_specs=[
            pl.BlockSpec(memory_space=pltpu.SMEM),
            pl.BlockSpec(memory_space=pltpu.VMEM),
        ],
        out_specs=pl.BlockSpec(memory_space=pltpu.SMEM),
        out_shape=jax.ShapeDtypeStruct((1, 1), jnp.float32),
    )(scalars, hc_all)


def kernel(prob_map_pred, thresh_map_pred, binary_map_pred,
           prob_map_gt, thresh_map_gt, mask_gt):
    hc_all = _sc_hist(prob_map_pred, prob_map_gt)
    scalars = _tc_dense(prob_map_pred, prob_map_gt, binary_map_pred,
                        thresh_map_pred, thresh_map_gt, mask_gt)
    out = _tc_finalize(scalars, hc_all)
    return out[0, 0]
